# baseline (speedup 1.0000x reference)
import functools

import jax
import jax.numpy as jnp
from jax import lax
from jax.experimental import pallas as pl
from jax.experimental.pallas import tpu as pltpu
from jax.experimental.pallas import tpu_sc as plsc

N = 10000
E = 320000
D_IN = 128
H = 64
W = 128

NC = 2
NS = 16
NW = NC * NS
BATCH = 128
NBUF = 2
EPT = -(-E // NW)
KCH = -(--(-EPT // BATCH) // NBUF) * NBUF
EPT_PAD = KCH * BATCH
N_PAD = -(-(N + 1) // (NS * 8)) * (NS * 8)
RPT = N_PAD // NS


def _edge_agg_body(y_hbm, e_hbm, agg_out,
                   e0, e1, b0, b1, acc_sh, g0, g1):
    c = lax.axis_index("c")
    s = lax.axis_index("s")
    wid = s * NC + c
    base = s * RPT
    eb = [e0, e1]
    bufs = [b0, b1]
    gsems = [g0, g1]

    zv = jnp.zeros((16,), jnp.float32)

    def zrow(r, _):
        def zcol(k, _):
            b0[r, pl.ds(k * 16, 16)] = zv
            return 0
        return lax.fori_loop(0, W // 16, zcol, 0)
    lax.fori_loop(0, BATCH, zrow, 0)

    nfull = RPT // BATCH
    rem = RPT % BATCH

    def zacc(i, _):
        pltpu.sync_copy(b0, acc_sh.at[pl.ds(base + i * BATCH, BATCH)])
        return 0
    lax.fori_loop(0, nfull, zacc, 0)
    if rem:
        pltpu.sync_copy(b0.at[pl.ds(0, rem)],
                        acc_sh.at[pl.ds(base + nfull * BATCH, rem)])

    for b in range(NBUF):
        pltpu.sync_copy(e_hbm.at[wid, b], eb[b])
        pltpu.async_copy(y_hbm.at[eb[b].at[0]], bufs[b], gsems[b])

    plsc.subcore_barrier()

    def chunk(i, _):
        j0 = i * NBUF
        for b in range(NBUF):
            pltpu.make_async_copy(y_hbm.at[eb[b].at[0]],
                                  bufs[b], gsems[b]).wait()
            pltpu.sync_copy(bufs[b], acc_sh.at[eb[b].at[1]], add=True)
            pltpu.sync_copy(e_hbm.at[wid, j0 + b + NBUF], eb[b])
            pltpu.async_copy(y_hbm.at[eb[b].at[0]], bufs[b], gsems[b])
        return 0
    lax.fori_loop(0, KCH // NBUF - 1, chunk, 0)

    for b in range(NBUF):
        pltpu.make_async_copy(y_hbm.at[eb[b].at[0]], bufs[b], gsems[b]).wait()
        pltpu.sync_copy(bufs[b], acc_sh.at[eb[b].at[1]], add=True)

    plsc.subcore_barrier()

    def wout(i, _):
        pltpu.sync_copy(acc_sh.at[pl.ds(base + i * BATCH, BATCH)], b0)
        pltpu.sync_copy(b0, agg_out.at[c, pl.ds(base + i * BATCH, BATCH)])
        return 0
    lax.fori_loop(0, nfull, wout, 0)
    if rem:
        pltpu.sync_copy(acc_sh.at[pl.ds(base + nfull * BATCH, rem)],
                        b0.at[pl.ds(0, rem)])
        pltpu.sync_copy(b0.at[pl.ds(0, rem)],
                        agg_out.at[c, pl.ds(base + nfull * BATCH, rem)])


_edge_agg = functools.partial(
    pl.kernel,
    mesh=plsc.VectorSubcoreMesh(core_axis_name="c", subcore_axis_name="s"),
    out_type=jax.ShapeDtypeStruct((NC, N_PAD, W), jnp.float32),
    scratch_types=(
        [pltpu.VMEM((2, BATCH), jnp.int32)] * NBUF
        + [pltpu.VMEM((BATCH, W), jnp.float32)] * NBUF
        + [pltpu.VMEM_SHARED((N_PAD, W), jnp.float32)]
        + [pltpu.SemaphoreType.DMA] * NBUF
    ),
)(_edge_agg_body)


def _msg_table(h, wl):
    y = jnp.dot(h, wl, preferred_element_type=jnp.float32)
    ones = jnp.ones((h.shape[0], 1), jnp.float32)
    zeros = jnp.zeros((h.shape[0], W - H - 1), jnp.float32)
    return jnp.concatenate([y, ones, zeros], axis=1)


def _tc_pre_body(x_ref, wl_ref, wr_ref, b_ref, y_ref, z_ref):
    x = x_ref[...]
    y_ref[...] = _msg_table(x, wl_ref[...])
    z_ref[...] = (jnp.dot(x, wr_ref[...], preferred_element_type=jnp.float32)
                  + b_ref[...])


def _mean_relu(aggp_ref, z_ref):
    agg = (aggp_ref[0] + aggp_ref[1])[:N]
    deg = agg[:, H]
    deginv = 1.0 / jnp.maximum(deg, 1.0)
    return jnp.maximum(agg[:, :H] * deginv[:, None] + z_ref[...], 0.0)


def _tc_mid_body(aggp_ref, z_ref, wl_ref, wr_ref, b_ref, y_ref, z2_ref):
    h1 = _mean_relu(aggp_ref, z_ref)
    y_ref[...] = _msg_table(h1, wl_ref[...])
    z2_ref[...] = (jnp.dot(h1, wr_ref[...], preferred_element_type=jnp.float32)
                   + b_ref[...])


def _tc_post_body(aggp_ref, z_ref, wc_ref, bc_ref, out_ref):
    h2 = _mean_relu(aggp_ref, z_ref)
    out_ref[...] = (jnp.dot(h2, wc_ref[...], preferred_element_type=jnp.float32)
                    + bc_ref[...])


_tc_pre = pl.pallas_call(
    _tc_pre_body,
    out_shape=[jax.ShapeDtypeStruct((N, W), jnp.float32),
               jax.ShapeDtypeStruct((N, H), jnp.float32)],
)

_tc_mid = pl.pallas_call(
    _tc_mid_body,
    out_shape=[jax.ShapeDtypeStruct((N, W), jnp.float32),
               jax.ShapeDtypeStruct((N, H), jnp.float32)],
)

_tc_post = pl.pallas_call(
    _tc_post_body,
    out_shape=jax.ShapeDtypeStruct((N, 1), jnp.float32),
)


def kernel(x, edge_index, W1l, W1r, b1, W2l, W2r, b2, Wc, bc):
    pad = NW * EPT_PAD - E
    src_i = jnp.concatenate([edge_index[0], jnp.zeros((pad,), jnp.int32)])
    dst_i = jnp.concatenate([edge_index[1], jnp.full((pad,), N, jnp.int32)])
    edges = jnp.stack([src_i.reshape(NW, KCH, BATCH),
                       dst_i.reshape(NW, KCH, BATCH)], axis=2)

    y1, z1 = _tc_pre(x, W1l.T, W1r.T, b1.reshape(1, H))
    aggp1 = _edge_agg(y1, edges)
    y2, z2 = _tc_mid(aggp1, z1, W2l.T, W2r.T, b2.reshape(1, H))
    aggp2 = _edge_agg(y2, edges)
    out = _tc_post(aggp2, z2, Wc.T, bc.reshape(1, 1))
    return out.reshape(N)


# speedup vs baseline: 1.0022x; 1.0022x over previous
import functools

import jax
import jax.numpy as jnp
from jax import lax
from jax.experimental import pallas as pl
from jax.experimental.pallas import tpu as pltpu
from jax.experimental.pallas import tpu_sc as plsc

N = 10000
E = 320000
D_IN = 128
H = 64
W = 128
SCW = W

NC = 2
NS = 16
NW = NC * NS
BATCH = 128
NBUF = 2
NIDX = 8
EPT = -(-E // NW)
KCH = -(--(-EPT // BATCH) // NIDX) * NIDX
EPT_PAD = KCH * BATCH
N_PAD = -(-(N + 1) // (NS * 8)) * (NS * 8)
RPT = N_PAD // NS


def _edge_agg_body(y_hbm, e_hbm, agg_out,
                   e0, e1, e2, e3, e4, e5, e6, e7, b0, b1, acc_sh,
                   g0, g1, i0, i1, i2, i3, i4, i5, i6, i7):
    c = lax.axis_index("c")
    s = lax.axis_index("s")
    wid = s * NC + c
    base = s * RPT
    eb = [e0, e1, e2, e3, e4, e5, e6, e7]
    isems = [i0, i1, i2, i3, i4, i5, i6, i7]
    bufs = [b0, b1]
    gsems = [g0, g1]

    zv = jnp.zeros((16,), jnp.float32)

    def zrow(r, _):
        def zcol(k, _):
            b0[r, pl.ds(k * 16, 16)] = zv
            return 0
        return lax.fori_loop(0, W // 16, zcol, 0)
    lax.fori_loop(0, BATCH, zrow, 0)

    nfull = RPT // BATCH
    rem = RPT % BATCH

    def zacc(i, _):
        pltpu.sync_copy(b0, acc_sh.at[pl.ds(base + i * BATCH, BATCH)])
        return 0
    lax.fori_loop(0, nfull, zacc, 0)
    if rem:
        pltpu.sync_copy(b0.at[pl.ds(0, rem)],
                        acc_sh.at[pl.ds(base + nfull * BATCH, rem)])

    for q in range(NBUF):
        pltpu.sync_copy(e_hbm.at[wid, q], eb[q])
    for b in range(NBUF):
        pltpu.async_copy(y_hbm.at[eb[b].at[0]], bufs[b], gsems[b])
    for q in range(NBUF, NIDX):
        pltpu.async_copy(e_hbm.at[wid, q], eb[q], isems[q])

    plsc.subcore_barrier()

    def octet(i, _):
        j0 = i * NIDX
        for k in range(NIDX):
            b = k % NBUF
            pltpu.make_async_copy(y_hbm.at[eb[k].at[0]],
                                  bufs[b], gsems[b]).wait()
            pltpu.sync_copy(bufs[b], acc_sh.at[eb[k].at[1]], add=True)
            pltpu.async_copy(e_hbm.at[wid, j0 + k + NIDX], eb[k], isems[k])
            kn = (k + NBUF) % NIDX
            pltpu.make_async_copy(e_hbm.at[wid, 0], eb[kn], isems[kn]).wait()
            pltpu.async_copy(y_hbm.at[eb[kn].at[0]], bufs[b], gsems[b])
        return 0
    lax.fori_loop(0, KCH // NIDX - 1, octet, 0)

    for k in range(NIDX):
        b = k % NBUF
        pltpu.make_async_copy(y_hbm.at[eb[k].at[0]], bufs[b], gsems[b]).wait()
        pltpu.sync_copy(bufs[b], acc_sh.at[eb[k].at[1]], add=True)
        if k < NIDX - NBUF:
            kn = k + NBUF
            pltpu.make_async_copy(e_hbm.at[wid, 0], eb[kn], isems[kn]).wait()
            pltpu.async_copy(y_hbm.at[eb[kn].at[0]], bufs[b], gsems[b])

    plsc.subcore_barrier()

    def wout(i, _):
        pltpu.sync_copy(acc_sh.at[pl.ds(base + i * BATCH, BATCH)], b0)
        pltpu.sync_copy(b0, agg_out.at[c, pl.ds(base + i * BATCH, BATCH)])
        return 0
    lax.fori_loop(0, nfull, wout, 0)
    if rem:
        pltpu.sync_copy(acc_sh.at[pl.ds(base + nfull * BATCH, rem)],
                        b0.at[pl.ds(0, rem)])
        pltpu.sync_copy(b0.at[pl.ds(0, rem)],
                        agg_out.at[c, pl.ds(base + nfull * BATCH, rem)])


_edge_agg = functools.partial(
    pl.kernel,
    mesh=plsc.VectorSubcoreMesh(core_axis_name="c", subcore_axis_name="s"),
    out_type=jax.ShapeDtypeStruct((NC, N_PAD, W), jnp.float32),
    scratch_types=(
        [pltpu.VMEM((2, BATCH), jnp.int32)] * NIDX
        + [pltpu.VMEM((BATCH, W), jnp.float32)] * NBUF
        + [pltpu.VMEM_SHARED((N_PAD, W), jnp.float32)]
        + [pltpu.SemaphoreType.DMA] * (NBUF + NIDX)
    ),
)(_edge_agg_body)


def _msg_table(h, wl):
    y = jnp.dot(h, wl, preferred_element_type=jnp.float32)
    ones = jnp.ones((h.shape[0], 1), jnp.float32)
    zeros = jnp.zeros((h.shape[0], W - H - 1), jnp.float32)
    return jnp.concatenate([y, ones, zeros], axis=1)


def _tc_pre_body(x_ref, wl_ref, wr_ref, b_ref, y_ref, z_ref):
    x = x_ref[...]
    y_ref[...] = _msg_table(x, wl_ref[...])
    z_ref[...] = (jnp.dot(x, wr_ref[...], preferred_element_type=jnp.float32)
                  + b_ref[...])


def _mean_relu(aggp_ref, z_ref):
    agg = (aggp_ref[0] + aggp_ref[1])[:N]
    deg = agg[:, H]
    deginv = 1.0 / jnp.maximum(deg, 1.0)
    return jnp.maximum(agg[:, :H] * deginv[:, None] + z_ref[...], 0.0)


def _tc_mid_body(aggp_ref, z_ref, wl_ref, wr_ref, b_ref, y_ref, z2_ref):
    h1 = _mean_relu(aggp_ref, z_ref)
    y_ref[...] = _msg_table(h1, wl_ref[...])
    z2_ref[...] = (jnp.dot(h1, wr_ref[...], preferred_element_type=jnp.float32)
                   + b_ref[...])


def _tc_post_body(aggp_ref, z_ref, wc_ref, bc_ref, out_ref):
    h2 = _mean_relu(aggp_ref, z_ref)
    out_ref[...] = (jnp.dot(h2, wc_ref[...], preferred_element_type=jnp.float32)
                    + bc_ref[...])


_tc_pre = pl.pallas_call(
    _tc_pre_body,
    out_shape=[jax.ShapeDtypeStruct((N, W), jnp.float32),
               jax.ShapeDtypeStruct((N, H), jnp.float32)],
)

_tc_mid = pl.pallas_call(
    _tc_mid_body,
    out_shape=[jax.ShapeDtypeStruct((N, W), jnp.float32),
               jax.ShapeDtypeStruct((N, H), jnp.float32)],
)

_tc_post = pl.pallas_call(
    _tc_post_body,
    out_shape=jax.ShapeDtypeStruct((N, 1), jnp.float32),
)


def kernel(x, edge_index, W1l, W1r, b1, W2l, W2r, b2, Wc, bc):
    pad = NW * EPT_PAD - E
    src_i = jnp.concatenate([edge_index[0], jnp.zeros((pad,), jnp.int32)])
    dst_i = jnp.concatenate([edge_index[1], jnp.full((pad,), N, jnp.int32)])
    edges = jnp.stack([src_i.reshape(NW, KCH, BATCH),
                       dst_i.reshape(NW, KCH, BATCH)], axis=2)

    y1, z1 = _tc_pre(x, W1l.T, W1r.T, b1.reshape(1, H))
    aggp1 = _edge_agg(y1, edges)
    y2, z2 = _tc_mid(aggp1, z1, W2l.T, W2r.T, b2.reshape(1, H))
    aggp2 = _edge_agg(y2, edges)
    out = _tc_post(aggp2, z2, Wc.T, bc.reshape(1, 1))
    return out.reshape(N)
